# Initial kernel scaffold; baseline (speedup 1.0000x reference)
import jax
import jax.numpy as jnp
from jax.experimental import pallas as pl


def kernel(x, embedding, start_token, end_token):
    raise NotImplementedError("write your pallas kernel here")


# speedup vs baseline: 1.2352x; 1.2352x over previous
import functools

import jax
import jax.numpy as jnp
import numpy as np
from jax import lax
from jax.experimental import pallas as pl
from jax.experimental.pallas import tpu as pltpu
from jax.experimental.pallas import tpu_sc as plsc

MAX_SEQ = 2048
D_MODEL = 2048
VOCAB = 77
BATCH = 4
LANES = 16


@functools.lru_cache(maxsize=1)
def _pe_np():
    position = np.arange(MAX_SEQ, dtype=np.float32)[:, None]
    div_term = np.exp(
        np.arange(0, D_MODEL, 2, dtype=np.float32) * -(np.log(10000.0) / D_MODEL)
    )
    pe = np.zeros((MAX_SEQ, D_MODEL), dtype=np.float32)
    pe[:, 0::2] = np.sin(position * div_term)
    pe[:, 1::2] = np.cos(position * div_term)
    return pe


@functools.lru_cache(maxsize=1)
def _make_sc_kernel():
    info = plsc.get_sparse_core_info()
    NC, NS = info.num_cores, info.num_subcores
    NW = NC * NS
    N = BATCH * MAX_SEQ
    TPW = N // NW
    K = 8
    NCH = TPW // K
    SPB = MAX_SEQ // TPW
    mesh = plsc.VectorSubcoreMesh(core_axis_name="c", subcore_axis_name="s")

    @functools.partial(
        pl.kernel,
        mesh=mesh,
        out_type=jax.ShapeDtypeStruct((N, D_MODEL), jnp.float32),
        scratch_types=[
            pltpu.VMEM((NCH, K), jnp.int32),
            pltpu.VMEM((K, D_MODEL), jnp.float32),
            pltpu.VMEM((K, D_MODEL), jnp.float32),
            pltpu.SemaphoreType.DMA,
            pltpu.SemaphoreType.DMA,
        ],
    )
    def k(x_hbm, table_hbm, pe_hbm, out_hbm, idx_v, emb_v, pe_v, gsem, psem):
        wid = lax.axis_index("s") * NC + lax.axis_index("c")
        pltpu.sync_copy(x_hbm.at[wid], idx_v)
        s_base = (wid % SPB) * TPW
        row0 = wid * TPW
        for c in range(NCH):
            g = pltpu.async_copy(table_hbm.at[idx_v.at[c]], emb_v, gsem)
            p = pltpu.async_copy(
                pe_hbm.at[pl.ds(s_base + c * K, K)], pe_v, psem
            )
            g.wait()
            p.wait()

            def body(i, _):
                col = i * LANES
                for r in range(K):
                    v = emb_v[r, pl.ds(col, LANES)]
                    plsc.addupdate(pe_v.at[r, pl.ds(col, LANES)], v)
                return 0

            lax.fori_loop(0, D_MODEL // LANES, body, 0)
            pltpu.sync_copy(pe_v, out_hbm.at[pl.ds(row0 + c * K, K)])

    return k


def kernel(x, embedding, start_token, end_token):
    del start_token, end_token
    info = plsc.get_sparse_core_info()
    NW = info.num_cores * info.num_subcores
    N = BATCH * MAX_SEQ
    TPW = N // NW
    K = 8
    x3 = x.astype(jnp.int32).reshape(NW, TPW // K, K)
    pe = jnp.asarray(_pe_np())
    out = _make_sc_kernel()(x3, embedding, pe)
    return out.reshape(BATCH, MAX_SEQ, D_MODEL)


# speedup vs baseline: 1.6054x; 1.2998x over previous
import functools

import jax
import jax.numpy as jnp
import numpy as np
from jax import lax
from jax.experimental import pallas as pl
from jax.experimental.pallas import tpu as pltpu
from jax.experimental.pallas import tpu_sc as plsc

MAX_SEQ = 2048
D_MODEL = 2048
VOCAB = 77
BATCH = 4
LANES = 16


@functools.lru_cache(maxsize=1)
def _pe_np():
    position = np.arange(MAX_SEQ, dtype=np.float32)[:, None]
    div_term = np.exp(
        np.arange(0, D_MODEL, 2, dtype=np.float32) * -(np.log(10000.0) / D_MODEL)
    )
    pe = np.zeros((MAX_SEQ, D_MODEL), dtype=np.float32)
    pe[:, 0::2] = np.sin(position * div_term)
    pe[:, 1::2] = np.cos(position * div_term)
    return pe


@functools.lru_cache(maxsize=1)
def _make_sc_kernel():
    info = plsc.get_sparse_core_info()
    NC, NS = info.num_cores, info.num_subcores
    NW = NC * NS
    N = BATCH * MAX_SEQ
    TPW = N // NW
    K = 8
    NCH = TPW // K
    SPB = MAX_SEQ // TPW
    mesh = plsc.VectorSubcoreMesh(core_axis_name="c", subcore_axis_name="s")

    @functools.partial(
        pl.kernel,
        mesh=mesh,
        out_type=jax.ShapeDtypeStruct((N, D_MODEL), jnp.float32),
        scratch_types=[
            pltpu.VMEM((NCH, K), jnp.int32),
            pltpu.VMEM((K, D_MODEL), jnp.float32),
            pltpu.VMEM((K, D_MODEL), jnp.float32),
            pltpu.VMEM((K, D_MODEL), jnp.float32),
            pltpu.VMEM((K, D_MODEL), jnp.float32),
            pltpu.SemaphoreType.DMA,
            pltpu.SemaphoreType.DMA,
            pltpu.SemaphoreType.DMA,
            pltpu.SemaphoreType.DMA,
            pltpu.SemaphoreType.DMA,
            pltpu.SemaphoreType.DMA,
        ],
    )
    def k(x_hbm, table_hbm, pe_hbm, out_hbm, idx_v,
          emb0, emb1, pe0, pe1, g0, g1, p0, p1, o0, o1):
        emb = (emb0, emb1)
        peb = (pe0, pe1)
        gs = (g0, g1)
        ps = (p0, p1)
        osem = (o0, o1)
        wid = lax.axis_index("s") * NC + lax.axis_index("c")
        pltpu.sync_copy(x_hbm.at[wid], idx_v)
        s_base = (wid % SPB) * TPW
        row0 = wid * TPW

        def start(c):
            b = c % 2
            gh = pltpu.async_copy(table_hbm.at[idx_v.at[c]], emb[b], gs[b])
            ph = pltpu.async_copy(
                pe_hbm.at[pl.ds(s_base + c * K, K)], peb[b], ps[b]
            )
            return gh, ph

        in_flight = {0: start(0)}
        out_flight = {}
        for c in range(NCH):
            b = c % 2
            if c + 1 < NCH:
                if c - 1 in out_flight:
                    out_flight.pop(c - 1).wait()
                in_flight[c + 1] = start(c + 1)
            gh, ph = in_flight.pop(c)
            gh.wait()
            ph.wait()

            def body(i, _):
                col = i * LANES
                for r in range(K):
                    v = emb[b][r, pl.ds(col, LANES)]
                    plsc.addupdate(peb[b].at[r, pl.ds(col, LANES)], v)
                return 0

            lax.fori_loop(0, D_MODEL // LANES, body, 0)
            out_flight[c] = pltpu.async_copy(
                peb[b], out_hbm.at[pl.ds(row0 + c * K, K)], osem[b]
            )
        for c in sorted(out_flight):
            out_flight.pop(c).wait()

    return k


def kernel(x, embedding, start_token, end_token):
    del start_token, end_token
    info = plsc.get_sparse_core_info()
    NW = info.num_cores * info.num_subcores
    N = BATCH * MAX_SEQ
    TPW = N // NW
    K = 8
    x3 = x.astype(jnp.int32).reshape(NW, TPW // K, K)
    pe = jnp.asarray(_pe_np())
    out = _make_sc_kernel()(x3, embedding, pe)
    return out.reshape(BATCH, MAX_SEQ, D_MODEL)


# speedup vs baseline: 1.6287x; 1.0145x over previous
import functools

import jax
import jax.numpy as jnp
import numpy as np
from jax import lax
from jax.experimental import pallas as pl
from jax.experimental.pallas import tpu as pltpu
from jax.experimental.pallas import tpu_sc as plsc

MAX_SEQ = 2048
D_MODEL = 2048
VOCAB = 77
BATCH = 4
LANES = 16


@functools.lru_cache(maxsize=1)
def _pe_np():
    position = np.arange(MAX_SEQ, dtype=np.float32)[:, None]
    div_term = np.exp(
        np.arange(0, D_MODEL, 2, dtype=np.float32) * -(np.log(10000.0) / D_MODEL)
    )
    pe = np.zeros((MAX_SEQ, D_MODEL), dtype=np.float32)
    pe[:, 0::2] = np.sin(position * div_term)
    pe[:, 1::2] = np.cos(position * div_term)
    return pe


@functools.lru_cache(maxsize=1)
def _make_sc_kernel():
    info = plsc.get_sparse_core_info()
    NC, NS = info.num_cores, info.num_subcores
    NW = NC * NS
    N = BATCH * MAX_SEQ
    TPW = N // NW
    K = 8
    NCH = TPW // K
    SPB = MAX_SEQ // TPW
    mesh = plsc.VectorSubcoreMesh(core_axis_name="c", subcore_axis_name="s")

    @functools.partial(
        pl.kernel,
        mesh=mesh,
        out_type=jax.ShapeDtypeStruct((N, D_MODEL), jnp.float32),
        scratch_types=[
            pltpu.VMEM((NCH, K), jnp.int32),
            pltpu.VMEM((K, D_MODEL), jnp.float32),
            pltpu.VMEM((K, D_MODEL), jnp.float32),
            pltpu.VMEM((K, D_MODEL), jnp.float32),
            pltpu.VMEM((K, D_MODEL), jnp.float32),
            pltpu.VMEM((K, D_MODEL), jnp.float32),
            pltpu.VMEM((K, D_MODEL), jnp.float32),
            pltpu.SemaphoreType.DMA,
            pltpu.SemaphoreType.DMA,
            pltpu.SemaphoreType.DMA,
            pltpu.SemaphoreType.DMA,
            pltpu.SemaphoreType.DMA,
            pltpu.SemaphoreType.DMA,
            pltpu.SemaphoreType.DMA,
            pltpu.SemaphoreType.DMA,
            pltpu.SemaphoreType.DMA,
        ],
    )
    def k(x_hbm, table_hbm, pe_hbm, out_hbm, idx_v,
          emb0, emb1, emb2, pe0, pe1, pe2,
          g0, g1, g2, p0, p1, p2, o0, o1, o2):
        emb = (emb0, emb1, emb2)
        peb = (pe0, pe1, pe2)
        gs = (g0, g1, g2)
        ps = (p0, p1, p2)
        osem = (o0, o1, o2)
        wid = lax.axis_index("s") * NC + lax.axis_index("c")
        pltpu.sync_copy(x_hbm.at[wid], idx_v)
        s_base = (wid % SPB) * TPW
        row0 = wid * TPW

        def start(c):
            b = c % 3
            gh = pltpu.async_copy(table_hbm.at[idx_v.at[c]], emb[b], gs[b])
            ph = pltpu.async_copy(
                pe_hbm.at[pl.ds(s_base + c * K, K)], peb[b], ps[b]
            )
            return gh, ph

        in_flight = {0: start(0), 1: start(1)}
        out_flight = {}
        for c in range(NCH):
            b = c % 3
            if c + 2 < NCH:
                if c - 1 in out_flight:
                    out_flight.pop(c - 1).wait()
                in_flight[c + 2] = start(c + 2)
            gh, ph = in_flight.pop(c)
            gh.wait()
            ph.wait()

            def body(i, _):
                col = i * LANES
                for r in range(K):
                    v = emb[b][r, pl.ds(col, LANES)]
                    plsc.addupdate(peb[b].at[r, pl.ds(col, LANES)], v)
                return 0

            lax.fori_loop(0, D_MODEL // LANES, body, 0)
            out_flight[c] = pltpu.async_copy(
                peb[b], out_hbm.at[pl.ds(row0 + c * K, K)], osem[b]
            )
        for c in sorted(out_flight):
            out_flight.pop(c).wait()

    return k


def kernel(x, embedding, start_token, end_token):
    del start_token, end_token
    info = plsc.get_sparse_core_info()
    NW = info.num_cores * info.num_subcores
    N = BATCH * MAX_SEQ
    TPW = N // NW
    K = 8
    x3 = x.astype(jnp.int32).reshape(NW, TPW // K, K)
    pe = jnp.asarray(_pe_np())
    out = _make_sc_kernel()(x3, embedding, pe)
    return out.reshape(BATCH, MAX_SEQ, D_MODEL)
